# baseline (speedup 1.0000x reference)
import functools

import jax
import jax.numpy as jnp
from jax import lax
from jax.experimental import pallas as pl
from jax.experimental.pallas import tpu as pltpu
from jax.experimental.pallas import tpu_sc as plsc


def _build_emb(H, B, V, D, NC, NS):
    NW = NC * NS
    BLK = B // NW
    CHUNK = BLK
    SLOTS = 4
    n_rounds = H // SLOTS
    assert H % SLOTS == 0 and BLK == 128

    mesh = plsc.VectorSubcoreMesh(core_axis_name="c", subcore_axis_name="s")

    @functools.partial(
        pl.kernel,
        mesh=mesh,
        out_type=jax.ShapeDtypeStruct((H, D // 8, NW, 8, BLK), jnp.float32),
        scratch_types=[
            pltpu.VMEM((H, BLK), jnp.int32),
            pltpu.VMEM((SLOTS, CHUNK, D), jnp.float32),
            pltpu.VMEM((SLOTS, D // 8, 1, 8, BLK), jnp.float32),
        ]
        + [pltpu.SemaphoreType.DMA] * (2 * SLOTS),
        compiler_params=pltpu.CompilerParams(
            use_tc_tiling_on_sc=False, needs_layout_passes=False
        ),
    )
    def emb(xt_hbm, table_hbm, out_hbm, idx_v, rows_v, tbuf_v, *sems):
        gsem = sems[:SLOTS]
        wsem = sems[SLOTS:]
        wid = lax.axis_index("s") * NC + lax.axis_index("c")
        base = wid * BLK
        pltpu.sync_copy(xt_hbm.at[:, pl.ds(base, BLK)], idx_v)

        lane = jax.lax.broadcasted_iota(jnp.int32, (16,), 0)
        rowvs = [lane + g * 16 for g in range(8)]

        def gdesc(h, j):
            return pltpu.make_async_copy(
                table_hbm.at[idx_v.at[h]], rows_v.at[j], gsem[j]
            )

        def wdesc(h, j):
            return pltpu.make_async_copy(
                tbuf_v.at[j],
                out_hbm.at[h, pl.ds(0, D // 8), pl.ds(wid, 1)],
                wsem[j],
            )

        for j in range(SLOTS):
            gdesc(j, j).start()

        def body(t, carry):
            for j in range(SLOTS):
                h = t * SLOTS + j
                gdesc(h, j).wait()

                @pl.when(t > 0)
                def _():
                    wdesc(h - SLOTS, j).wait()

                @plsc.parallel_loop(0, D, step=1, unroll=8)
                def trans(d):
                    colv = lane * 0 + d
                    dhi = d // 8
                    dlo = d % 8
                    for g in range(8):
                        vals = plsc.load_gather(
                            rows_v.at[j], [rowvs[g], colv]
                        )
                        tbuf_v[j, dhi, 0, dlo, pl.ds(g * 16, 16)] = vals

                wdesc(h, j).start()

                @pl.when(h + SLOTS < H)
                def _():
                    gdesc(h + SLOTS, j).start()

            return carry

        lax.fori_loop(0, n_rounds, body, 0)
        for j in range(SLOTS):
            wdesc(H - SLOTS + j, j).wait()

    return emb


def kernel(x, table):
    Bt, H = x.shape
    V, D = table.shape
    info = plsc.get_sparse_core_info()
    emb = _build_emb(H, Bt, V, D, info.num_cores, info.num_subcores)
    out5 = emb(x.T, table)
    return out5.transpose(2, 4, 0, 1, 3).reshape(Bt, H, D)


# speedup vs baseline: 1.7096x; 1.7096x over previous
import functools

import jax
import jax.numpy as jnp
from jax import lax
from jax.experimental import pallas as pl
from jax.experimental.pallas import tpu as pltpu
from jax.experimental.pallas import tpu_sc as plsc


def _build_emb(H, B, V, D, NC, NS):
    NW = NC * NS
    BLK = B // NW
    CHUNK = BLK
    SLOTS = 4
    n_rounds = H // SLOTS
    assert H % SLOTS == 0 and BLK == 128

    mesh = plsc.VectorSubcoreMesh(core_axis_name="c", subcore_axis_name="s")

    @functools.partial(
        pl.kernel,
        mesh=mesh,
        out_type=jax.ShapeDtypeStruct((H, D // 8, NW, 8, BLK), jnp.float32),
        scratch_types=[
            pltpu.VMEM((H, BLK), jnp.int32),
            pltpu.VMEM((SLOTS, CHUNK, D), jnp.float32),
            pltpu.VMEM((SLOTS, D // 8, 1, 8, BLK + 5), jnp.float32),
        ]
        + [pltpu.SemaphoreType.DMA] * (2 * SLOTS),
        compiler_params=pltpu.CompilerParams(
            use_tc_tiling_on_sc=False, needs_layout_passes=False
        ),
    )
    def emb(xt_hbm, table_hbm, out_hbm, idx_v, rows_v, tbuf_v, *sems):
        gsem = sems[:SLOTS]
        wsem = sems[SLOTS:]
        wid = lax.axis_index("s") * NC + lax.axis_index("c")
        base = wid * BLK
        pltpu.sync_copy(xt_hbm.at[:, pl.ds(base, BLK)], idx_v)

        lane = jax.lax.broadcasted_iota(jnp.int32, (16,), 0)
        lane_hi = lane // 8
        lane_lo = lane % 8
        zeros16 = lane * 0
        dhis = [lane_hi + 2 * k for k in range(D // 16)]

        def gdesc(h, j):
            return pltpu.make_async_copy(
                table_hbm.at[idx_v.at[h]], rows_v.at[j], gsem[j]
            )

        def wdesc(h, j):
            return pltpu.make_async_copy(
                tbuf_v.at[j, :, :, :, pl.ds(0, BLK)],
                out_hbm.at[h, pl.ds(0, D // 8), pl.ds(wid, 1)],
                wsem[j],
            )

        for j in range(SLOTS):
            gdesc(j, j).start()

        def body(t, carry):
            for j in range(SLOTS):
                h = t * SLOTS + j
                gdesc(h, j).wait()

                @pl.when(t > 0)
                def _():
                    wdesc(h - SLOTS, j).wait()

                @plsc.parallel_loop(0, CHUNK, step=1, unroll=8)
                def trans(b):
                    bfull = zeros16 + b
                    for k in range(D // 16):
                        vals = rows_v[j, b, pl.ds(k * 16, 16)]
                        plsc.store_scatter(
                            tbuf_v.at[j],
                            [dhis[k], zeros16, lane_lo, bfull],
                            vals,
                        )

                wdesc(h, j).start()

                @pl.when(h + SLOTS < H)
                def _():
                    gdesc(h + SLOTS, j).start()

            return carry

        lax.fori_loop(0, n_rounds, body, 0)
        for j in range(SLOTS):
            wdesc(H - SLOTS + j, j).wait()

    return emb


def kernel(x, table):
    Bt, H = x.shape
    V, D = table.shape
    info = plsc.get_sparse_core_info()
    emb = _build_emb(H, Bt, V, D, info.num_cores, info.num_subcores)
    out5 = emb(x.T, table)
    return out5.transpose(2, 4, 0, 1, 3).reshape(Bt, H, D)
